# baseline (speedup 1.0000x reference)
import functools

import jax
import jax.numpy as jnp
from jax.experimental import pallas as pl


def _mm_body(p_ref, w_ref, b_ref, o_ref, *, act):
    acc = jnp.dot(p_ref[...], w_ref[...], preferred_element_type=jnp.float32)
    acc = acc + b_ref[...]
    if act == "relu":
        acc = jnp.maximum(acc, 0.0)
    elif act == "sigmoid":
        acc = jax.nn.sigmoid(acc)
    o_ref[...] = acc


def _matmul(p, w, b, act=None, mb=2048):
    m, k = p.shape
    n = w.shape[1]
    assert m % mb == 0, (m, mb)
    return pl.pallas_call(
        functools.partial(_mm_body, act=act),
        grid=(m // mb,),
        in_specs=[
            pl.BlockSpec((mb, k), lambda i: (i, 0)),
            pl.BlockSpec((k, n), lambda i: (0, 0)),
            pl.BlockSpec((1, n), lambda i: (0, 0)),
        ],
        out_specs=pl.BlockSpec((mb, n), lambda i: (i, 0)),
        out_shape=jax.ShapeDtypeStruct((m, n), jnp.float32),
    )(p, w, b.reshape(1, n))


def _vq_body(z_ref, c_ref, idx_ref, zq_ref, loss_ref):
    z = z_ref[...]
    c = c_ref[...]
    s = jax.lax.dot_general(z, c, (((1,), (1,)), ((), ())),
                            preferred_element_type=jnp.float32)
    z2 = jnp.sum(z * z, axis=1, keepdims=True)
    c2 = jnp.sum(c * c, axis=1)[None, :]
    dists = z2 + c2 - 2.0 * s
    m = jnp.min(dists, axis=1, keepdims=True)
    iota = jax.lax.broadcasted_iota(jnp.int32, dists.shape, 1)
    idx = jnp.min(jnp.where(dists <= m, iota, dists.shape[1]), axis=1)
    idx_ref[0, 0, :] = idx
    oh = (iota == idx[:, None]).astype(jnp.float32)
    zq = jax.lax.dot_general(oh, c, (((1,), (0,)), ((), ())),
                             preferred_element_type=jnp.float32)
    zq_ref[...] = zq
    part = jnp.sum((zq - z) ** 2).reshape(1, 1)

    @pl.when(pl.program_id(0) == 0)
    def _():
        loss_ref[...] = jnp.zeros((1, 1), jnp.float32)

    loss_ref[...] += part


def _vq(z_flat, codebook, mb=2048):
    m, d = z_flat.shape
    v = codebook.shape[0]
    g = m // mb
    idx3, zq, loss = pl.pallas_call(
        _vq_body,
        grid=(g,),
        in_specs=[
            pl.BlockSpec((mb, d), lambda i: (i, 0)),
            pl.BlockSpec((v, d), lambda i: (0, 0)),
        ],
        out_specs=[
            pl.BlockSpec((1, 1, mb), lambda i: (i, 0, 0)),
            pl.BlockSpec((mb, d), lambda i: (i, 0)),
            pl.BlockSpec((1, 1), lambda i: (0, 0)),
        ],
        out_shape=[
            jax.ShapeDtypeStruct((g, 1, mb), jnp.int32),
            jax.ShapeDtypeStruct((m, d), jnp.float32),
            jax.ShapeDtypeStruct((1, 1), jnp.float32),
        ],
    )(z_flat, codebook)
    return idx3.reshape(m), zq, loss[0, 0]


def _im2col_s1k3(x):
    b, h, w, c = x.shape
    xp = jnp.pad(x, ((0, 0), (1, 1), (1, 1), (0, 0)))
    cols = [xp[:, th:th + h, tw:tw + w, :] for th in range(3) for tw in range(3)]
    return jnp.concatenate(cols, axis=-1).reshape(b * h * w, 9 * c)


def _im2col_s2k4(x):
    b, h, w, c = x.shape
    ho, wo = h // 2, w // 2
    xp = jnp.pad(x, ((0, 0), (1, 1), (1, 1), (0, 0)))
    cols = [xp[:, th:th + 2 * ho:2, tw:tw + 2 * wo:2, :]
            for th in range(4) for tw in range(4)]
    return jnp.concatenate(cols, axis=-1).reshape(b * ho * wo, 16 * c)


_PHASE_TAPS = {0: ((0, 0), (1, 2)), 1: ((1, 1), (2, 3))}


def _convt_weights(w):
    i, o = w.shape[0], w.shape[1]
    wf = jnp.flip(w, axis=(2, 3)).transpose(1, 0, 2, 3)
    wf = wf.transpose(2, 3, 1, 0)
    wc = jnp.zeros((3, 3, i, 2, 2, o), jnp.float32)
    for r in (0, 1):
        for s in (0, 1):
            for ph, dh in _PHASE_TAPS[r]:
                for pw, dw in _PHASE_TAPS[s]:
                    wc = wc.at[ph, pw, :, r, s, :].set(wf[dh, dw])
    return wc.reshape(9 * i, 4 * o)


def _interleave(y, b, h, w, c):
    y = y.reshape(b, h, w, 2, 2, c).transpose(0, 1, 3, 2, 4, 5)
    return y.reshape(b, 2 * h, 2 * w, c)


def kernel(x, enc_w1, enc_b1, enc_w2, enc_b2, enc_w3, enc_b3, codebook,
           dec_w1, dec_b1, dec_w2, dec_b2, dec_w3, dec_b3):
    B = x.shape[0]

    p1 = _im2col_s2k4(x.transpose(0, 2, 3, 1))
    w1m = enc_w1.transpose(2, 3, 1, 0).reshape(16, 32)
    h1 = _matmul(p1, w1m, enc_b1, "relu")

    p2 = _im2col_s2k4(h1.reshape(B, 112, 112, 32))
    w2m = enc_w2.transpose(2, 3, 1, 0).reshape(512, 64)
    h2 = _matmul(p2, w2m, enc_b2, "relu")

    p3 = _im2col_s1k3(h2.reshape(B, 56, 56, 64))
    w3m = enc_w3.transpose(2, 3, 1, 0).reshape(576, 64)
    z_flat = _matmul(p3, w3m, enc_b3)

    idx, zq_flat, sse = _vq(z_flat, codebook)
    q_loss = sse / z_flat.size
    vq_loss = q_loss + 0.25 * q_loss
    z_q_st = zq_flat.reshape(B, 56, 56, 64).transpose(0, 3, 1, 2)

    wd1 = jnp.flip(dec_w1, axis=(2, 3)).transpose(1, 0, 2, 3)
    wd1m = wd1.transpose(2, 3, 1, 0).reshape(576, 64)
    pd1 = _im2col_s1k3(zq_flat.reshape(B, 56, 56, 64))
    r1 = _matmul(pd1, wd1m, dec_b1, "relu")

    wc2 = _convt_weights(dec_w2)
    pd2 = _im2col_s1k3(r1.reshape(B, 56, 56, 64))
    o2 = _matmul(pd2, wc2, jnp.tile(dec_b2, 4), "relu")
    r2 = _interleave(o2, B, 56, 56, 32)

    wc3 = _convt_weights(dec_w3)
    pd3 = _im2col_s1k3(r2)
    o3 = _matmul(pd3, wc3, jnp.tile(dec_b3, 4), "sigmoid")
    x_recon = _interleave(o3, B, 112, 112, 1).transpose(0, 3, 1, 2)

    return x_recon, z_q_st, idx, vq_loss


# speedup vs baseline: 20.3716x; 20.3716x over previous
import functools

import jax
import jax.numpy as jnp
from jax.experimental import pallas as pl

_G = 58
_M = _G * _G
_OFF = 64
_S = 3488


def _conv_body(in_ref, w_ref, b_ref, o_ref, *, act):
    taps = [in_ref[pl.ds(_OFF + (dh - 1) * _G + (dw - 1), _M), :]
            for dh in range(3) for dw in range(3)]
    p = jnp.concatenate(taps, axis=1)
    acc = jnp.dot(p, w_ref[...], preferred_element_type=jnp.float32)
    acc = acc + b_ref[...]
    if act == "relu":
        acc = jnp.maximum(acc, 0.0)
    elif act == "sigmoid":
        acc = jax.nn.sigmoid(acc)
    q = jax.lax.broadcasted_iota(jnp.int32, (_M, 1), 0)
    r, c = q // _G, q % _G
    valid = (r >= 1) & (r <= 56) & (c >= 1) & (c <= 56)
    acc = jnp.where(valid, acc, 0.0)
    o_ref[pl.ds(0, _OFF), :] = jnp.zeros((_OFF, acc.shape[1]), jnp.float32)
    o_ref[pl.ds(_OFF, _M), :] = acc
    o_ref[pl.ds(_S - 64, 64), :] = jnp.zeros((64, acc.shape[1]), jnp.float32)


def _conv(x, w, b, act=None):
    bm, cin = x.shape
    nb = bm // _S
    k, n = w.shape
    return pl.pallas_call(
        functools.partial(_conv_body, act=act),
        grid=(nb,),
        in_specs=[
            pl.BlockSpec((_S, cin), lambda i: (i, 0)),
            pl.BlockSpec((k, n), lambda i: (0, 0)),
            pl.BlockSpec((1, n), lambda i: (0, 0)),
        ],
        out_specs=pl.BlockSpec((_S, n), lambda i: (i, 0)),
        out_shape=jax.ShapeDtypeStruct((bm, n), jnp.float32),
    )(x, w, b.reshape(1, n))


def _vq_body(z_ref, c_ref, idx_ref, zq_ref, loss_ref):
    z = z_ref[...]
    c = c_ref[...]
    s = jax.lax.dot_general(z, c, (((1,), (1,)), ((), ())),
                            preferred_element_type=jnp.float32)
    z2 = jnp.sum(z * z, axis=1, keepdims=True)
    c2 = jnp.sum(c * c, axis=1)[None, :]
    dists = z2 + c2 - 2.0 * s
    m = jnp.min(dists, axis=1, keepdims=True)
    iota = jax.lax.broadcasted_iota(jnp.int32, dists.shape, 1)
    idx = jnp.min(jnp.where(dists <= m, iota, dists.shape[1]), axis=1)
    idx_ref[0, 0, :] = idx
    oh = (iota == idx[:, None]).astype(jnp.float32)
    zq = jax.lax.dot_general(oh, c, (((1,), (0,)), ((), ())),
                             preferred_element_type=jnp.float32)
    q = jax.lax.broadcasted_iota(jnp.int32, (_S, 1), 0) - _OFF
    r, cc = q // _G, q % _G
    valid = (q >= 0) & (q < _M) & (r >= 1) & (r <= 56) & (cc >= 1) & (cc <= 56)
    zq = jnp.where(valid, zq, 0.0)
    zq_ref[...] = zq
    part = jnp.sum((zq - z) ** 2).reshape(1, 1)

    @pl.when(pl.program_id(0) == 0)
    def _():
        loss_ref[...] = jnp.zeros((1, 1), jnp.float32)

    loss_ref[...] += part


def _vq(z_flat, codebook):
    bm, d = z_flat.shape
    v = codebook.shape[0]
    g = bm // _S
    idx3, zq, loss = pl.pallas_call(
        _vq_body,
        grid=(g,),
        in_specs=[
            pl.BlockSpec((_S, d), lambda i: (i, 0)),
            pl.BlockSpec((v, d), lambda i: (0, 0)),
        ],
        out_specs=[
            pl.BlockSpec((1, 1, _S), lambda i: (i, 0, 0)),
            pl.BlockSpec((_S, d), lambda i: (i, 0)),
            pl.BlockSpec((1, 1), lambda i: (0, 0)),
        ],
        out_shape=[
            jax.ShapeDtypeStruct((g, 1, _S), jnp.int32),
            jax.ShapeDtypeStruct((bm, d), jnp.float32),
            jax.ShapeDtypeStruct((1, 1), jnp.float32),
        ],
    )(z_flat, codebook)
    return idx3, zq, loss[0, 0]


def _fold4(v):
    return v // 4, v % 4


def _w_enc1(w):
    o = w.shape[0]
    ws = jnp.zeros((3, 3, 4, 4, 2, 2, o), jnp.float32)
    for fh in range(2):
        for th in range(4):
            dh, eh = _fold4(2 * fh + th - 1)
            for fw in range(2):
                for tw in range(4):
                    dw, ew = _fold4(2 * fw + tw - 1)
                    ws = ws.at[dh + 1, dw + 1, eh, ew, fh, fw, :].set(w[:, 0, th, tw])
    return ws.reshape(9 * 16, 4 * o)


def _w_enc2(w):
    o, i = w.shape[0], w.shape[1]
    ws = jnp.zeros((3, 3, 2, 2, i, o), jnp.float32)
    for th in range(4):
        dh, fh = (th - 1) // 2, (th - 1) % 2
        for tw in range(4):
            dw, fw = (tw - 1) // 2, (tw - 1) % 2
            ws = ws.at[dh + 1, dw + 1, fh, fw, :, :].set(w[:, :, th, tw].T)
    return ws.reshape(9 * 4 * i, o)


def _w_s1(w):
    return w.transpose(2, 3, 1, 0).reshape(-1, w.shape[0])


_PHASE_TAPS = {0: ((0, 0), (1, 2)), 1: ((1, 1), (2, 3))}


def _w_dec2(w):
    i, o = w.shape[0], w.shape[1]
    wf = jnp.flip(w, axis=(2, 3)).transpose(1, 0, 2, 3).transpose(2, 3, 1, 0)
    wc = jnp.zeros((3, 3, i, 2, 2, o), jnp.float32)
    for r in (0, 1):
        for s in (0, 1):
            for ph, dh in _PHASE_TAPS[r]:
                for pw, dw in _PHASE_TAPS[s]:
                    wc = wc.at[ph, pw, :, r, s, :].set(wf[dh, dw])
    return wc.reshape(9 * i, 4 * o)


def _w_dec3(w):
    i = w.shape[0]
    wf = jnp.flip(w, axis=(2, 3)).transpose(1, 0, 2, 3).transpose(2, 3, 1, 0)
    ws = jnp.zeros((3, 3, 2, 2, i, 4, 4), jnp.float32)
    for fh in range(2):
        for th in range(4):
            dh, eh = _fold4(2 * fh - th + 2)
            for fw in range(2):
                for tw in range(4):
                    dw, ew = _fold4(2 * fw - tw + 2)
                    ws = ws.at[1 - dh, 1 - dw, fh, fw, :, eh, ew].set(wf[th, tw, :, 0])
    return ws.reshape(9 * 4 * i, 16)


def _to_grid(x):
    b, _, _, c = x.shape
    xp = jnp.pad(x, ((0, 0), (1, 1), (1, 1), (0, 0))).reshape(b, _M, c)
    xp = jnp.pad(xp, ((0, 0), (_OFF, _S - _OFF - _M), (0, 0)))
    return xp.reshape(b * _S, c)


def _from_grid(x, b):
    c = x.shape[1]
    xg = x.reshape(b, _S, c)[:, _OFF:_OFF + _M, :].reshape(b, _G, _G, c)
    return xg[:, 1:57, 1:57, :]


def kernel(x, enc_w1, enc_b1, enc_w2, enc_b2, enc_w3, enc_b3, codebook,
           dec_w1, dec_b1, dec_w2, dec_b2, dec_w3, dec_b3):
    B = x.shape[0]

    xs = x.reshape(B, 56, 4, 56, 4).transpose(0, 1, 3, 2, 4).reshape(B, 56, 56, 16)
    xs = _to_grid(xs)

    h1 = _conv(xs, _w_enc1(enc_w1), jnp.tile(enc_b1, 4), "relu")
    h2 = _conv(h1, _w_enc2(enc_w2), enc_b2, "relu")
    z = _conv(h2, _w_s1(enc_w3), enc_b3)

    idx3, zq, sse = _vq(z, codebook)
    q_loss = sse / (B * 56 * 56 * 64)
    vq_loss = q_loss + 0.25 * q_loss

    idxg = idx3.reshape(B, _S)[:, _OFF:_OFF + _M].reshape(B, _G, _G)
    idx = idxg[:, 1:57, 1:57].reshape(B * 56 * 56)
    z_q_st = _from_grid(zq, B).transpose(0, 3, 1, 2)

    wd1 = jnp.flip(dec_w1, axis=(2, 3)).transpose(1, 0, 2, 3)
    r1 = _conv(zq, _w_s1(wd1), dec_b1, "relu")
    r2 = _conv(r1, _w_dec2(dec_w2), jnp.tile(dec_b2, 4), "relu")
    xr = _conv(r2, _w_dec3(dec_w3), jnp.tile(dec_b3, 16), "sigmoid")

    xr = _from_grid(xr, B).reshape(B, 56, 56, 4, 4)
    x_recon = xr.transpose(0, 1, 3, 2, 4).reshape(B, 1, 224, 224)

    return x_recon, z_q_st, idx, vq_loss
